# baseline (speedup 1.0000x reference)
import jax
import jax.numpy as jnp
from jax.experimental import pallas as pl
from jax.experimental.pallas import tpu as pltpu

_N = 2048
_R = 8
_B = 4
_H = 64
_O = 32
_NEG = 0.2

_TR1 = 256
_SUB1 = 2
_SR1 = _TR1 // _SUB1
_NSLOT1 = 2 * _R
_NI1 = _N // _TR1
_T1 = _NI1 * _R

_TR2 = 512
_SUB2 = 2
_SR2 = _TR2 // _SUB2
_NSLOT2 = 8
_NI2 = _N // _TR2
_T2 = _NI2 * _B


def _combine_kernel(wr_ref, wb_ref, out_ref):
    for r in range(_R):
        acc = wr_ref[r, 0] * wb_ref[0]
        for b in range(1, _B):
            acc = acc + wr_ref[r, b] * wb_ref[b]
        out_ref[r] = acc


def _z_kernel(x_ref, wb_ref, z_ref):
    x = x_ref[:]
    for b in range(_B):
        z_ref[b] = jnp.dot(x, wb_ref[b], preferred_element_type=jnp.float32)


def _leaky(v):
    return jnp.where(v >= 0, v, _NEG * v)


def _l2norm(v):
    n = jnp.sqrt(jnp.sum(v * v, axis=1, keepdims=True))
    return v / jnp.maximum(n, 1e-12)


def _pass1_kernel(a_ref, w1_ref, wr2_ref, x_ref, ab_ref,
                  buf_ref, sem, conv_ref, wsem):
    def start_reads(tile, slot):
        i = tile // _R
        r = tile % _R
        for q in range(_SUB1):
            pltpu.make_async_copy(
                a_ref.at[r, pl.ds(i * _TR1 + q * _SR1, _SR1), :],
                buf_ref.at[pl.ds(slot * _TR1 + q * _SR1, _SR1), :],
                sem.at[slot, q],
            ).start()

    t = pl.program_id(0)

    @pl.when(t == 0)
    def _():
        for j in range(_R):
            start_reads(j, j)

    nxt = t + _R

    @pl.when(nxt < _T1)
    def _():
        start_reads(nxt, nxt % _NSLOT1)

    slot = t % _NSLOT1
    i = t // _R
    r = t % _R
    for q in range(_SUB1):
        pltpu.make_async_copy(
            a_ref.at[0, pl.ds(0, _SR1), :],
            buf_ref.at[pl.ds(0, _SR1), :],
            sem.at[slot, q],
        ).wait()

    tile = buf_ref[pl.ds(slot * _TR1, _TR1), :]
    contrib = jnp.dot(tile, w1_ref[r], preferred_element_type=jnp.float32)
    sl = pl.ds(i * _TR1, _TR1)

    @pl.when(r == 0)
    def _():
        x_ref[sl, :] = contrib

    @pl.when(r > 0)
    def _():
        x_ref[sl, :] = x_ref[sl, :] + contrib

    @pl.when(r == _R - 1)
    def _():
        x_ref[sl, :] = _leaky(x_ref[sl, :])

        @pl.when(i > 0)
        def _():
            pltpu.make_async_copy(
                conv_ref, ab_ref.at[:, pl.ds(0, _TR1), :], wsem
            ).wait()

        base = (i % 2) * (_R * _TR1)
        for b in range(_B):
            plane = wr2_ref[0, b] * buf_ref[pl.ds(base, _TR1), :]
            for j in range(1, _R):
                plane = plane + wr2_ref[j, b] * buf_ref[
                    pl.ds(base + j * _TR1, _TR1), :
                ]
            conv_ref[b] = plane.astype(jnp.bfloat16)
        pltpu.make_async_copy(
            conv_ref, ab_ref.at[:, sl, :], wsem
        ).start()

    @pl.when(t == _T1 - 1)
    def _():
        pltpu.make_async_copy(
            conv_ref, ab_ref.at[:, pl.ds(0, _TR1), :], wsem
        ).wait()


def _pass2_kernel(ab_ref, z_ref, out_ref, buf_ref, sem):
    def start_reads(tile, slot):
        i = tile // _B
        b = tile % _B
        for q in range(_SUB2):
            pltpu.make_async_copy(
                ab_ref.at[b, pl.ds(i * _TR2 + q * _SR2, _SR2), :],
                buf_ref.at[pl.ds(slot * _TR2 + q * _SR2, _SR2), :],
                sem.at[slot, q],
            ).start()

    t = pl.program_id(0)

    @pl.when(t == 0)
    def _():
        for j in range(_NSLOT2 - 1):
            start_reads(j, j)

    nxt = t + _NSLOT2 - 1

    @pl.when(nxt < _T2)
    def _():
        start_reads(nxt, nxt % _NSLOT2)

    slot = t % _NSLOT2
    i = t // _B
    b = t % _B
    for q in range(_SUB2):
        pltpu.make_async_copy(
            ab_ref.at[0, pl.ds(0, _SR2), :],
            buf_ref.at[pl.ds(0, _SR2), :],
            sem.at[slot, q],
        ).wait()

    tile = buf_ref[pl.ds(slot * _TR2, _TR2), :].astype(jnp.float32)
    contrib = jnp.dot(tile, z_ref[b], preferred_element_type=jnp.float32)
    sl = pl.ds(i * _TR2, _TR2)

    @pl.when(b == 0)
    def _():
        out_ref[sl, :] = contrib

    @pl.when(b > 0)
    def _():
        out_ref[sl, :] = out_ref[sl, :] + contrib

    @pl.when(b == _B - 1)
    def _():
        out_ref[sl, :] = _l2norm(out_ref[sl, :])


@jax.jit
def kernel(A, X, w_bases1, w_rel1, w_bases2, w_rel2):
    del X
    w1 = pl.pallas_call(
        _combine_kernel,
        out_shape=jax.ShapeDtypeStruct((_R, _N, _H), jnp.float32),
        in_specs=[
            pl.BlockSpec(memory_space=pltpu.SMEM),
            pl.BlockSpec(memory_space=pltpu.MemorySpace.VMEM),
        ],
        out_specs=pl.BlockSpec(memory_space=pltpu.MemorySpace.VMEM),
    )(w_rel1, w_bases1)

    x, abar2 = pl.pallas_call(
        _pass1_kernel,
        grid=(_T1,),
        in_specs=[
            pl.BlockSpec(memory_space=pltpu.MemorySpace.HBM),
            pl.BlockSpec((_R, _N, _H), lambda t: (0, 0, 0)),
            pl.BlockSpec(memory_space=pltpu.SMEM),
        ],
        out_specs=[
            pl.BlockSpec((_N, _H), lambda t: (0, 0)),
            pl.BlockSpec(memory_space=pltpu.MemorySpace.HBM),
        ],
        out_shape=[
            jax.ShapeDtypeStruct((_N, _H), jnp.float32),
            jax.ShapeDtypeStruct((_B, _N, _N), jnp.bfloat16),
        ],
        scratch_shapes=[
            pltpu.VMEM((_NSLOT1 * _TR1, _N), jnp.float32),
            pltpu.SemaphoreType.DMA((_NSLOT1, _SUB1)),
            pltpu.VMEM((_B, _TR1, _N), jnp.bfloat16),
            pltpu.SemaphoreType.DMA,
        ],
        compiler_params=pltpu.CompilerParams(
            dimension_semantics=("arbitrary",),
        ),
    )(A, w1, w_rel2)

    z = pl.pallas_call(
        _z_kernel,
        out_shape=jax.ShapeDtypeStruct((_B, _N, _O), jnp.float32),
        in_specs=[
            pl.BlockSpec(memory_space=pltpu.MemorySpace.VMEM),
            pl.BlockSpec(memory_space=pltpu.MemorySpace.VMEM),
        ],
        out_specs=pl.BlockSpec(memory_space=pltpu.MemorySpace.VMEM),
    )(x, w_bases2)

    out = pl.pallas_call(
        _pass2_kernel,
        grid=(_T2,),
        in_specs=[
            pl.BlockSpec(memory_space=pltpu.MemorySpace.HBM),
            pl.BlockSpec((_B, _N, _O), lambda t: (0, 0, 0)),
        ],
        out_specs=pl.BlockSpec((_N, _O), lambda t: (0, 0)),
        out_shape=jax.ShapeDtypeStruct((_N, _O), jnp.float32),
        scratch_shapes=[
            pltpu.VMEM((_NSLOT2 * _TR2, _N), jnp.bfloat16),
            pltpu.SemaphoreType.DMA((_NSLOT2, _SUB2)),
        ],
        compiler_params=pltpu.CompilerParams(
            dimension_semantics=("arbitrary",),
        ),
    )(abar2, z)
    return out
